# baseline (speedup 1.0000x reference)
import functools

import jax
import jax.numpy as jnp
from jax import lax
from jax.experimental import pallas as pl
from jax.experimental.pallas import tpu as pltpu
from jax.experimental.pallas import tpu_sc as plsc

SEQ = 2048
EMB = 1024
BATCH = 4


def _tc_add(x, pos_table, nb, blk):
    B, S, E = x.shape

    def body(x_ref, p_ref, o_ref):
        o_ref[...] = x_ref[...] + p_ref[...]

    return pl.pallas_call(
        body,
        grid=(S // blk, nb),
        in_specs=[
            pl.BlockSpec((1, blk, E), lambda s, b: (b, s, 0)),
            pl.BlockSpec((blk, E), lambda s, b: (s, 0)),
        ],
        out_specs=pl.BlockSpec((1, blk, E), lambda s, b: (b, s, 0)),
        out_shape=jax.ShapeDtypeStruct((nb, S, E), x.dtype),
    )(x, pos_table)


def _sc_add(x, p, b0, nb):
    info = plsc.get_sparse_core_info()
    NC, NS, L = info.num_cores, info.num_subcores, info.num_lanes
    NW = NC * NS
    rows_per_w = SEQ // NW
    R = 16
    n_chunks = rows_per_w // R
    n_steps = n_chunks * nb

    mesh = plsc.VectorSubcoreMesh(core_axis_name="c", subcore_axis_name="s")

    NBUF = 3

    @functools.partial(
        pl.kernel,
        mesh=mesh,
        out_type=jax.ShapeDtypeStruct((nb, SEQ, EMB), jnp.float32),
        scratch_types=[
            pltpu.VMEM((NBUF, R, EMB), jnp.float32),
            pltpu.VMEM((2, R, EMB), jnp.float32),
            pltpu.VMEM((NBUF, R, EMB), jnp.float32),
            pltpu.SemaphoreType.DMA((NBUF,)),
            pltpu.SemaphoreType.DMA((NBUF,)),
            pltpu.SemaphoreType.DMA((2,)),
        ],
    )
    def k(x_hbm, p_hbm, o_hbm, x_v, pos_v, o_v, in_sem, out_sem, p_sem):
        wid = lax.axis_index("s") * NC + lax.axis_index("c")
        row0 = wid * rows_per_w

        def start_x_in(step):
            buf = step % NBUF
            c, b = divmod(step, nb)
            return pltpu.async_copy(
                x_hbm.at[b0 + b, pl.ds(row0 + c * R, R)], x_v.at[buf],
                in_sem.at[buf])

        def start_out(step):
            buf = step % NBUF
            c, b = divmod(step, nb)
            return pltpu.async_copy(
                o_v.at[buf], o_hbm.at[b, pl.ds(row0 + c * R, R)],
                out_sem.at[buf])

        def start_pos_in(c):
            buf = c % 2
            return pltpu.async_copy(
                p_hbm.at[pl.ds(row0 + c * R, R)], pos_v.at[buf],
                p_sem.at[buf])

        pos_h = {0: start_pos_in(0)}
        if n_chunks > 1:
            pos_h[1] = start_pos_in(1)
        x_h = {}
        for s in range(min(NBUF - 1, n_steps)):
            x_h[s] = start_x_in(s)
        out_h = {}
        for s in range(n_steps):
            buf = s % NBUF
            c, b = divmod(s, nb)
            if s + NBUF - 1 < n_steps:
                x_h[s + NBUF - 1] = start_x_in(s + NBUF - 1)
            if b == 0 and c >= 1 and c + 1 < n_chunks:
                pos_h[c + 1] = start_pos_in(c + 1)
            x_h.pop(s).wait()
            if b == 0 and c in pos_h:
                pos_h.pop(c).wait()
            if (s - NBUF) in out_h:
                out_h.pop(s - NBUF).wait()

            @plsc.parallel_loop(0, R)
            def row_body(r, _buf=buf, _pb=c % 2):
                @plsc.parallel_loop(0, EMB // L, unroll=16)
                def col_body(i, _r=r):
                    off = i * L
                    o_v[_buf, _r, pl.ds(off, L)] = (
                        x_v[_buf, _r, pl.ds(off, L)]
                        + pos_v[_pb, _r, pl.ds(off, L)])
            out_h[s] = start_out(s)
        for h in out_h.values():
            h.wait()

    return k(x, p)


def kernel(x, pos_table):
    return _sc_add(x, pos_table, 0, BATCH)


# speedup vs baseline: 1.0289x; 1.0289x over previous
import functools

import jax
import jax.numpy as jnp
from jax import lax
from jax.experimental import pallas as pl
from jax.experimental.pallas import tpu as pltpu
from jax.experimental.pallas import tpu_sc as plsc

SEQ = 2048
EMB = 1024
BATCH = 4


def _tc_add(x, pos_table, nb, blk):
    B, S, E = x.shape

    def body(x_ref, p_ref, o_ref):
        o_ref[...] = x_ref[...] + p_ref[...]

    return pl.pallas_call(
        body,
        grid=(S // blk, nb),
        in_specs=[
            pl.BlockSpec((1, blk, E), lambda s, b: (b, s, 0)),
            pl.BlockSpec((blk, E), lambda s, b: (s, 0)),
        ],
        out_specs=pl.BlockSpec((1, blk, E), lambda s, b: (b, s, 0)),
        out_shape=jax.ShapeDtypeStruct((B, S, E), x.dtype),
    )(x, pos_table)


def _sc_add(x, p, b0, nb):
    info = plsc.get_sparse_core_info()
    NC, NS, L = info.num_cores, info.num_subcores, info.num_lanes
    NW = NC * NS
    rows_per_w = SEQ // NW
    R = 16
    n_chunks = rows_per_w // R
    n_steps = n_chunks * nb

    mesh = plsc.VectorSubcoreMesh(core_axis_name="c", subcore_axis_name="s")

    NBUF = 3

    @functools.partial(
        pl.kernel,
        mesh=mesh,
        out_type=jax.ShapeDtypeStruct((nb, SEQ, EMB), jnp.float32),
        scratch_types=[
            pltpu.VMEM((NBUF, R, EMB), jnp.float32),
            pltpu.VMEM((2, R, EMB), jnp.float32),
            pltpu.VMEM((NBUF, R, EMB), jnp.float32),
            pltpu.SemaphoreType.DMA((NBUF,)),
            pltpu.SemaphoreType.DMA((NBUF,)),
            pltpu.SemaphoreType.DMA((2,)),
        ],
    )
    def k(x_hbm, p_hbm, o_hbm, x_v, pos_v, o_v, in_sem, out_sem, p_sem):
        wid = lax.axis_index("s") * NC + lax.axis_index("c")
        row0 = wid * rows_per_w

        def start_x_in(step):
            buf = step % NBUF
            c, b = divmod(step, nb)
            return pltpu.async_copy(
                x_hbm.at[b0 + b, pl.ds(row0 + c * R, R)], x_v.at[buf],
                in_sem.at[buf])

        def start_out(step):
            buf = step % NBUF
            c, b = divmod(step, nb)
            return pltpu.async_copy(
                o_v.at[buf], o_hbm.at[b, pl.ds(row0 + c * R, R)],
                out_sem.at[buf])

        def start_pos_in(c):
            buf = c % 2
            return pltpu.async_copy(
                p_hbm.at[pl.ds(row0 + c * R, R)], pos_v.at[buf],
                p_sem.at[buf])

        pos_h = {0: start_pos_in(0)}
        if n_chunks > 1:
            pos_h[1] = start_pos_in(1)
        x_h = {}
        for s in range(min(NBUF - 1, n_steps)):
            x_h[s] = start_x_in(s)
        out_h = {}
        for s in range(n_steps):
            buf = s % NBUF
            c, b = divmod(s, nb)
            if s + NBUF - 1 < n_steps:
                x_h[s + NBUF - 1] = start_x_in(s + NBUF - 1)
            if b == 0 and c >= 1 and c + 1 < n_chunks:
                pos_h[c + 1] = start_pos_in(c + 1)
            x_h.pop(s).wait()
            if b == 0 and c in pos_h:
                pos_h.pop(c).wait()
            if (s - NBUF) in out_h:
                out_h.pop(s - NBUF).wait()

            @plsc.parallel_loop(0, R)
            def row_body(r, _buf=buf, _pb=c % 2):
                @plsc.parallel_loop(0, EMB // L, unroll=8)
                def col_body(i, _r=r):
                    off = i * L
                    o_v[_buf, _r, pl.ds(off, L)] = (
                        x_v[_buf, _r, pl.ds(off, L)]
                        + pos_v[_pb, _r, pl.ds(off, L)])
            out_h[s] = start_out(s)
        for h in out_h.values():
            h.wait()

    return k(x, p)


def kernel(x, pos_table):
    return _sc_add(x, pos_table, 0, BATCH)
